# baseline (speedup 1.0000x reference)
import functools

import jax
import jax.numpy as jnp
from jax import lax
from jax.experimental import pallas as pl
from jax.experimental.pallas import tpu as pltpu
from jax.experimental.pallas import tpu_sc as plsc

L = 16


@functools.lru_cache(maxsize=None)
def _build(M, K, D, NC, NS):
    NW = NC * NS
    B = 4
    G = B * K
    S = M // B
    NITER = ((S + NW - 1) // NW + 7) // 8 * 8
    NBUF = 4
    NPAD = (NITER + NBUF - 1) // NBUF * NBUF
    mesh = plsc.VectorSubcoreMesh(core_axis_name="c", subcore_axis_name="s")

    scratch = (
        [pltpu.VMEM((G,), jnp.int32) for _ in range(NBUF)]
        + [pltpu.VMEM((G, D), jnp.float32) for _ in range(NBUF)]
        + [pltpu.VMEM((B, D), jnp.float32) for _ in range(NBUF)]
        + [pltpu.SemaphoreType.DMA for _ in range(3 * NBUF)]
    )

    @functools.partial(
        pl.kernel,
        out_type=jax.ShapeDtypeStruct((M, D), jnp.float32),
        mesh=mesh,
        scratch_types=scratch,
    )
    def pool(points_hbm, idx_hbm, out_hbm, *bufs):
        idxb = bufs[0:NBUF]
        rows = bufs[NBUF:2 * NBUF]
        outs = bufs[2 * NBUF:3 * NBUF]
        isem = bufs[3 * NBUF:4 * NBUF]
        gsem = bufs[4 * NBUF:5 * NBUF]
        ssem = bufs[5 * NBUF:6 * NBUF]
        wid = lax.axis_index("s") * NC + lax.axis_index("c")
        lo = wid * NITER
        n_valid = jnp.minimum(NITER, S - lo)
        inv = jnp.float32(1.0 / K)

        def valid(j):
            return j < n_valid

        def idx_load(j, b):
            return pltpu.make_async_copy(
                idx_hbm.at[pl.ds((lo + j) * G, G)], idxb[b], isem[b])

        def gather(j, b):
            return pltpu.make_async_copy(
                points_hbm.at[idxb[b]], rows[b], gsem[b])

        def store(j, b):
            return pltpu.make_async_copy(
                outs[b], out_hbm.at[pl.ds((lo + j) * B, B)], ssem[b])

        for b in range(NBUF - 1):
            @pl.when(valid(b))
            def _(b=b):
                idx_load(b, b).start()
                idx_load(b, b).wait()
                gather(b, b).start()

        @pl.when(valid(NBUF - 1))
        def _():
            idx_load(NBUF - 1, NBUF - 1).start()

        def step(j, b):
            j2 = j + NBUF - 1
            b2 = (b + NBUF - 1) % NBUF

            @pl.when(valid(j2))
            def _():
                idx_load(j2, b2).wait()
                gather(j2, b2).start()

            @pl.when(valid(j))
            def _():
                gather(j, b).wait()

                @pl.when(valid(j + NBUF))
                def _():
                    idx_load(j + NBUF, b).start()

                @pl.when(j >= NBUF)
                def _():
                    store(j - NBUF, b).wait()

                NCH = D // L
                KU = 16
                for r in range(B):
                    zero = jnp.zeros((L,), jnp.float32)

                    def kbody(t, accs, r=r):
                        base = r * K + t * KU
                        new = []
                        for ch in range(NCH):
                            sl = pl.ds(ch * L, L)
                            a = accs[ch]
                            for u in range(KU):
                                a = a + rows[b][base + u, sl]
                            new.append(a)
                        return tuple(new)

                    accs = lax.fori_loop(0, K // KU, kbody, (zero,) * NCH)
                    for ch in range(NCH):
                        outs[b][r, pl.ds(ch * L, L)] = accs[ch] * inv
                store(j, b).start()

        @pl.loop(0, NPAD, step=NBUF)
        def _(g):
            for b in range(NBUF):
                step(g + b, b)

        for b in range(NBUF):
            @pl.when(n_valid > b)
            def _(b=b):
                store(b, b).wait()

    def run(points, idx_flat):
        pad = NW * NITER * G - idx_flat.shape[0]
        idx_padded = jnp.pad(idx_flat, (0, pad)) if pad else idx_flat
        return pool(points, idx_padded)

    return run


def kernel(points, indices):
    M, D = points.shape
    K = indices.shape[1]
    info = plsc.get_sparse_core_info()
    idx_flat = indices.astype(jnp.int32).reshape(-1)
    return _build(M, K, D, info.num_cores, info.num_subcores)(points, idx_flat)


# speedup vs baseline: 1.0392x; 1.0392x over previous
import functools

import jax
import jax.numpy as jnp
from jax import lax
from jax.experimental import pallas as pl
from jax.experimental.pallas import tpu as pltpu
from jax.experimental.pallas import tpu_sc as plsc

L = 16


@functools.lru_cache(maxsize=None)
def _build(M, K, D, NC, NS):
    NW = NC * NS
    G = 128
    BR = G // K
    B = 2 * BR
    S = M // B
    NITER = ((S + NW - 1) // NW + 7) // 8 * 8
    NBUF = 3
    NPAD = (NITER + NBUF - 1) // NBUF * NBUF
    mesh = plsc.VectorSubcoreMesh(core_axis_name="c", subcore_axis_name="s")

    scratch = (
        [pltpu.VMEM((G,), jnp.int32) for _ in range(2 * NBUF)]
        + [pltpu.VMEM((G, D), jnp.float32) for _ in range(2 * NBUF)]
        + [pltpu.VMEM((B, D), jnp.float32) for _ in range(NBUF)]
        + [pltpu.SemaphoreType.DMA for _ in range(5 * NBUF)]
    )

    @functools.partial(
        pl.kernel,
        out_type=jax.ShapeDtypeStruct((M, D), jnp.float32),
        mesh=mesh,
        scratch_types=scratch,
    )
    def pool(points_hbm, idx_hbm, out_hbm, *bufs):
        idxA = bufs[0:NBUF]
        idxB = bufs[NBUF:2 * NBUF]
        rowsA = bufs[2 * NBUF:3 * NBUF]
        rowsB = bufs[3 * NBUF:4 * NBUF]
        outs = bufs[4 * NBUF:5 * NBUF]
        isemA = bufs[5 * NBUF:6 * NBUF]
        isemB = bufs[6 * NBUF:7 * NBUF]
        gsemA = bufs[7 * NBUF:8 * NBUF]
        gsemB = bufs[8 * NBUF:9 * NBUF]
        ssem = bufs[9 * NBUF:10 * NBUF]
        wid = lax.axis_index("s") * NC + lax.axis_index("c")
        lo = wid * NITER
        n_valid = jnp.minimum(NITER, S - lo)
        inv = jnp.float32(1.0 / K)

        def valid(j):
            return j < n_valid

        def idx_load(j, b, half):
            idx = (idxA, idxB)[half][b]
            sem = (isemA, isemB)[half][b]
            off = (lo + j) * 2 * G + half * G
            return pltpu.make_async_copy(idx_hbm.at[pl.ds(off, G)], idx, sem)

        def gather(j, b, half):
            idx = (idxA, idxB)[half][b]
            dst = (rowsA, rowsB)[half][b]
            sem = (gsemA, gsemB)[half][b]
            return pltpu.make_async_copy(points_hbm.at[idx], dst, sem)

        def store(j, b):
            return pltpu.make_async_copy(
                outs[b], out_hbm.at[pl.ds((lo + j) * B, B)], ssem[b])

        for b in range(NBUF - 1):
            @pl.when(valid(b))
            def _(b=b):
                for half in (0, 1):
                    idx_load(b, b, half).start()
                    idx_load(b, b, half).wait()
                    gather(b, b, half).start()

        @pl.when(valid(NBUF - 1))
        def _():
            for half in (0, 1):
                idx_load(NBUF - 1, NBUF - 1, half).start()

        def step(j, b):
            j2 = j + NBUF - 1
            b2 = (b + NBUF - 1) % NBUF

            @pl.when(valid(j2))
            def _():
                for half in (0, 1):
                    idx_load(j2, b2, half).wait()
                    gather(j2, b2, half).start()

            @pl.when(valid(j))
            def _():
                gather(j, b, 0).wait()
                gather(j, b, 1).wait()

                @pl.when(valid(j + NBUF))
                def _():
                    for half in (0, 1):
                        idx_load(j + NBUF, b, half).start()

                @pl.when(j >= NBUF)
                def _():
                    store(j - NBUF, b).wait()

                NCH = D // L
                KU = 8
                for r in range(B):
                    src = rowsA[b] if r < BR else rowsB[b]
                    rr = r % BR
                    zero = jnp.zeros((L,), jnp.float32)

                    def kbody(t, accs, src=src, rr=rr):
                        base = rr * K + t * KU
                        new = []
                        for ch in range(NCH):
                            sl = pl.ds(ch * L, L)
                            a = accs[ch]
                            for u in range(KU):
                                a = a + src[base + u, sl]
                            new.append(a)
                        return tuple(new)

                    accs = lax.fori_loop(0, K // KU, kbody, (zero,) * NCH)
                    for ch in range(NCH):
                        outs[b][r, pl.ds(ch * L, L)] = accs[ch] * inv
                store(j, b).start()

        @pl.loop(0, NPAD, step=NBUF)
        def _(g):
            for b in range(NBUF):
                step(g + b, b)

        for b in range(NBUF):
            @pl.when(n_valid > b)
            def _(b=b):
                store(b, b).wait()

    def run(points, idx_flat):
        pad = NW * NITER * 2 * G - idx_flat.shape[0]
        idx_padded = jnp.pad(idx_flat, (0, pad)) if pad else idx_flat
        return pool(points, idx_padded)

    return run


def kernel(points, indices):
    M, D = points.shape
    K = indices.shape[1]
    info = plsc.get_sparse_core_info()
    idx_flat = indices.astype(jnp.int32).reshape(-1)
    return _build(M, K, D, info.num_cores, info.num_subcores)(points, idx_flat)


# speedup vs baseline: 1.1648x; 1.1209x over previous
import functools

import jax
import jax.numpy as jnp
from jax import lax
from jax.experimental import pallas as pl
from jax.experimental.pallas import tpu as pltpu
from jax.experimental.pallas import tpu_sc as plsc

L = 16


@functools.lru_cache(maxsize=None)
def _build(M, K, D, NC, NS):
    NW = NC * NS
    B = 4
    G = B * K
    S = M // B
    NITER = ((S + NW - 1) // NW + 7) // 8 * 8
    NBUF = 3
    NPAD = (NITER + NBUF - 1) // NBUF * NBUF
    mesh = plsc.VectorSubcoreMesh(core_axis_name="c", subcore_axis_name="s")

    scratch = (
        [pltpu.VMEM((G,), jnp.int32) for _ in range(NBUF)]
        + [pltpu.VMEM((G, D), jnp.float32) for _ in range(NBUF)]
        + [pltpu.VMEM((B, D), jnp.float32) for _ in range(NBUF)]
        + [pltpu.SemaphoreType.DMA for _ in range(3 * NBUF)]
    )

    @functools.partial(
        pl.kernel,
        out_type=jax.ShapeDtypeStruct((M, D), jnp.float32),
        mesh=mesh,
        scratch_types=scratch,
    )
    def pool(points_hbm, idx_hbm, out_hbm, *bufs):
        idxb = bufs[0:NBUF]
        rows = bufs[NBUF:2 * NBUF]
        outs = bufs[2 * NBUF:3 * NBUF]
        isem = bufs[3 * NBUF:4 * NBUF]
        gsem = bufs[4 * NBUF:5 * NBUF]
        ssem = bufs[5 * NBUF:6 * NBUF]
        wid = lax.axis_index("s") * NC + lax.axis_index("c")
        lo = wid * NITER
        n_valid = jnp.minimum(NITER, S - lo)
        inv = jnp.float32(1.0 / K)

        def valid(j):
            return j < n_valid

        def idx_load(j, b):
            return pltpu.make_async_copy(
                idx_hbm.at[pl.ds((lo + j) * G, G)], idxb[b], isem[b])

        def gather(j, b):
            return pltpu.make_async_copy(
                points_hbm.at[idxb[b]], rows[b], gsem[b])

        def store(j, b):
            return pltpu.make_async_copy(
                outs[b], out_hbm.at[pl.ds((lo + j) * B, B)], ssem[b])

        for b in range(NBUF - 1):
            @pl.when(valid(b))
            def _(b=b):
                idx_load(b, b).start()
                idx_load(b, b).wait()
                gather(b, b).start()

        @pl.when(valid(NBUF - 1))
        def _():
            idx_load(NBUF - 1, NBUF - 1).start()

        def step(j, b):
            j2 = j + NBUF - 1
            b2 = (b + NBUF - 1) % NBUF

            @pl.when(valid(j2))
            def _():
                idx_load(j2, b2).wait()
                gather(j2, b2).start()

            @pl.when(valid(j))
            def _():
                gather(j, b).wait()

                @pl.when(valid(j + NBUF))
                def _():
                    idx_load(j + NBUF, b).start()

                @pl.when(j >= NBUF)
                def _():
                    store(j - NBUF, b).wait()

                NCH = D // L
                KU = 8
                for r in range(B):
                    zero = jnp.zeros((L,), jnp.float32)

                    def kbody(t, accs, r=r):
                        base = r * K + t * KU
                        new = []
                        for ch in range(NCH):
                            sl = pl.ds(ch * L, L)
                            a = accs[ch]
                            for u in range(KU):
                                a = a + rows[b][base + u, sl]
                            new.append(a)
                        return tuple(new)

                    accs = lax.fori_loop(0, K // KU, kbody, (zero,) * NCH)
                    for ch in range(NCH):
                        outs[b][r, pl.ds(ch * L, L)] = accs[ch] * inv
                store(j, b).start()

        @pl.loop(0, NPAD, step=NBUF)
        def _(g):
            for b in range(NBUF):
                step(g + b, b)

        for b in range(NBUF):
            @pl.when(n_valid > b)
            def _(b=b):
                store(b, b).wait()

    def run(points, idx_flat):
        pad = NW * NITER * G - idx_flat.shape[0]
        idx_padded = jnp.pad(idx_flat, (0, pad)) if pad else idx_flat
        return pool(points, idx_padded)

    return run


def kernel(points, indices):
    M, D = points.shape
    K = indices.shape[1]
    info = plsc.get_sparse_core_info()
    idx_flat = indices.astype(jnp.int32).reshape(-1)
    return _build(M, K, D, info.num_cores, info.num_subcores)(points, idx_flat)
